# baseline (speedup 1.0000x reference)
import functools

import jax
import jax.numpy as jnp
from jax import lax
from jax.experimental import pallas as pl
from jax.experimental.pallas import tpu as pltpu
from jax.experimental.pallas import tpu_sc as plsc

_NC = 2
_NS = 16
_NW = _NC * _NS
_D = 64
_JT = 7812
_PAIRS = 123
_RING = 32


@jax.jit
def _gather(labels, tblT, tail):
    batch = labels.shape[0]
    nblk = batch // 8
    mesh = plsc.VectorSubcoreMesh(core_axis_name="c", subcore_axis_name="s")

    @functools.partial(
        pl.kernel,
        out_type=jax.ShapeDtypeStruct((nblk, 8, _D), jnp.float32),
        mesh=mesh,
        scratch_types=[
            pltpu.VMEM((batch,), jnp.int32),
            pltpu.VMEM((batch + 16,), jnp.int32),
            pltpu.VMEM((batch + 16,), jnp.int32),
            pltpu.VMEM((_D, 128), jnp.float32),
            pltpu.VMEM((_D, 128), jnp.float32),
            pltpu.VMEM((_D, _D), jnp.float32),
            pltpu.VMEM((_RING, _D), jnp.float32),
            pltpu.SMEM((4,), jnp.int32),
            pltpu.SemaphoreType.DMA,
            pltpu.SemaphoreType.DMA,
            pltpu.SemaphoreType.DMA,
        ],
        compiler_params=pltpu.CompilerParams(needs_layout_passes=False),
    )
    def k(tbl_hbm, lab_hbm, tail_hbm, out_hbm, lab_v, ml_v, mp_v, b0_v, b1_v,
          tail_v, ring_v, ctr, sem0, sem1, semo):
        wid = lax.axis_index("s") * _NC + lax.axis_index("c")
        lo = wid * (_JT + 1) // _NW
        hi = (wid + 1) * (_JT + 1) // _NW
        hi_s = jnp.minimum(hi, _JT)
        iota = lax.iota(jnp.int32, 16)

        pltpu.sync_copy(lab_hbm, lab_v)
        pltpu.sync_copy(tail_hbm, tail_v)

        def scan_body(g, off):
            labv = lab_v[pl.ds(g * 16, 16)]
            jv = lax.shift_right_logical(labv, 7)
            m = jnp.logical_and(jv >= lo, jv < hi)
            plsc.store_compressed(ml_v.at[pl.ds(off, 16)], labv, mask=m)
            plsc.store_compressed(mp_v.at[pl.ds(off, 16)], g * 16 + iota, mask=m)
            return off + plsc.all_reduce_population_count(m)[0]

        n_mine = lax.fori_loop(0, batch // 16, scan_body, jnp.int32(0))
        ml_v[pl.ds(n_mine, 16)] = jnp.full((16,), jnp.int32(0x7FFFFFFF))
        ctr[0] = n_mine
        ctr[1] = 0
        nv = lax.shift_right_logical(n_mine + 15, 4)

        def start_chunk(j, buf, sem):
            off = pl.multiple_of(j * 128, 128)
            pltpu.make_async_copy(
                tbl_hbm.at[:, pl.ds(off, 128)], buf, sem
            ).start()

        def wait_chunk(buf, sem):
            pltpu.make_async_copy(tbl_hbm.at[:, pl.ds(0, 128)], buf, sem).wait()

        def emit_row(lab_i, pos_i, src_kind, buf):
            rc = ctr[1]
            slot = lax.bitwise_and(rc, _RING - 1)
            if src_kind == "stream":
                c = lax.bitwise_and(lab_i, 127)
                cv = jnp.broadcast_to(c, (16,))
                for kk in range(4):
                    val = plsc.load_gather(buf, [iota + kk * 16, cv])
                    ring_v[slot, pl.ds(kk * 16, 16)] = val
            else:
                r0 = jnp.broadcast_to(lab_i - (_JT * 128), (16,))
                for kk in range(4):
                    val = plsc.load_gather(tail_v, [r0, iota + kk * 16])
                    ring_v[slot, pl.ds(kk * 16, 16)] = val
            pltpu.make_async_copy(
                ring_v.at[slot],
                out_hbm.at[lax.shift_right_logical(pos_i, 3),
                           lax.bitwise_and(pos_i, 7)],
                semo,
            ).start()
            rc = rc + 1
            ctr[1] = rc

            @pl.when(lax.bitwise_and(rc, _RING - 1) == 0)
            def _():
                def dbody(_, x):
                    pltpu.make_async_copy(
                        ring_v.at[0], out_hbm.at[0, 0], semo
                    ).wait()
                    return x

                lax.fori_loop(0, _RING, dbody, 0)

        def process(j, buf, tail_mode):
            def pbody(g, _):
                mv = ml_v[pl.ds(g * 16, 16)]
                pv = mp_v[pl.ds(g * 16, 16)]
                jm = lax.shift_right_logical(mv, 7)
                m = jm == j
                mi = m.astype(jnp.int32)
                any_m = plsc.all_reduce_population_count(m)[0] > 0

                @pl.when(any_m)
                def _():
                    for i in range(16):
                        @pl.when(mi[i] != 0)
                        def _():
                            emit_row(mv[i], pv[i],
                                     "tail" if tail_mode else "stream", buf)
                return 0

            lax.fori_loop(0, nv, pbody, 0)

        start_chunk(lo, b0_v, sem0)

        @pl.when(lo + 1 < hi_s)
        def _():
            start_chunk(lo + 1, b1_v, sem1)

        def stream_body(t, _):
            j0 = lo + 2 * t
            j1 = j0 + 1

            @pl.when(j0 < hi_s)
            def _():
                wait_chunk(b0_v, sem0)
                process(j0, b0_v, False)

                @pl.when(j0 + 2 < hi_s)
                def _():
                    start_chunk(j0 + 2, b0_v, sem0)

            @pl.when(j1 < hi_s)
            def _():
                wait_chunk(b1_v, sem1)
                process(j1, b1_v, False)

                @pl.when(j1 + 2 < hi_s)
                def _():
                    start_chunk(j1 + 2, b1_v, sem1)

            return 0

        lax.fori_loop(0, _PAIRS, stream_body, 0)

        @pl.when(hi == _JT + 1)
        def _():
            process(jnp.int32(_JT), b0_v, True)

        rem = lax.bitwise_and(ctr[1], _RING - 1)

        def drain_body(_, x):
            pltpu.make_async_copy(ring_v.at[0], out_hbm.at[0, 0], semo).wait()
            return x

        lax.fori_loop(0, rem, drain_body, 0)

    return k(tblT, labels, tail)


def kernel(batch_size, class_labels, class_embedding):
    labels = class_labels.astype(jnp.int32)
    tblT = class_embedding.T
    tail = class_embedding[_JT * 128:]
    out = _gather(labels, tblT, tail)
    return out.reshape(-1, _D)


# speedup vs baseline: 8.1478x; 8.1478x over previous
import functools

import jax
import jax.numpy as jnp
from jax import lax
from jax.experimental import pallas as pl
from jax.experimental.pallas import tpu as pltpu
from jax.experimental.pallas import tpu_sc as plsc

_NC = 2
_NS = 16
_NW = _NC * _NS


@jax.jit
def _gather(labels, tbl3):
    batch = labels.shape[0]
    b_per_w = batch // _NW
    dim = tbl3.shape[2]
    mesh = plsc.VectorSubcoreMesh(core_axis_name="c", subcore_axis_name="s")

    @functools.partial(
        pl.kernel,
        out_type=jax.ShapeDtypeStruct((batch, dim), jnp.float32),
        mesh=mesh,
        scratch_types=[
            pltpu.VMEM((b_per_w,), jnp.int32),
            pltpu.VMEM((b_per_w, dim), jnp.float32),
            pltpu.SemaphoreType.DMA,
        ],
        compiler_params=pltpu.CompilerParams(needs_layout_passes=False),
    )
    def k(tbl_hbm, lab_hbm, out_hbm, lab_v, rows_v, sem):
        wid = lax.axis_index("s") * _NC + lax.axis_index("c")
        base = wid * b_per_w
        pltpu.sync_copy(lab_hbm.at[pl.ds(base, b_per_w)], lab_v)

        copies = []
        for g in range(b_per_w // 16):
            labv = lab_v[pl.ds(g * 16, 16)]
            for i in range(16):
                lab = labv[i]
                blk = lax.shift_right_logical(lab, 3)
                sel = lax.bitwise_and(lab, 7)
                copies.append(
                    pltpu.async_copy(
                        tbl_hbm.at[blk, sel], rows_v.at[g * 16 + i], sem
                    )
                )
        for c in copies:
            c.wait()
        pltpu.sync_copy(rows_v, out_hbm.at[pl.ds(base, b_per_w)])

    return k(tbl3, labels)


def kernel(batch_size, class_labels, class_embedding):
    labels = class_labels.astype(jnp.int32)
    tbl3 = class_embedding.reshape(-1, 8, class_embedding.shape[1])
    return _gather(labels, tbl3)
